# baseline (speedup 1.0000x reference)
import jax
import jax.numpy as jnp
from jax.experimental import pallas as pl
from jax.experimental.pallas import tpu as pltpu


def _mul_body(theta_ref, noise_ref, out_ref):
    sig = jnp.exp(theta_ref[...])
    out_ref[...] = noise_ref[...] * sig[..., None]


def kernel(concepts, embeddings, thetas_table):
    B, L = concepts.shape
    D = embeddings.shape[-1]
    thetas = jnp.take(thetas_table[:, 0], concepts, axis=0)
    noise = jax.random.normal(jax.random.key(42), (B, L, D), jnp.float32)
    BB = 32
    out = pl.pallas_call(
        _mul_body,
        grid=(B // BB,),
        in_specs=[
            pl.BlockSpec((BB, L), lambda i: (i, 0)),
            pl.BlockSpec((BB, L, D), lambda i: (i, 0, 0)),
        ],
        out_specs=pl.BlockSpec((BB, L, D), lambda i: (i, 0, 0)),
        out_shape=jax.ShapeDtypeStruct((B, L, D), jnp.float32),
    )(thetas, noise)
    return out


# speedup vs baseline: 1.5605x; 1.5605x over previous
import numpy as np
import jax
import jax.numpy as jnp
from jax import lax
from jax.experimental import pallas as pl
from jax.experimental.pallas import tpu as pltpu

_ROT_A = (13, 15, 26, 6)
_ROT_B = (17, 29, 16, 24)
_KS2 = 466688986 ^ 42
_KS = (np.int32(0), np.int32(42), np.int32(_KS2 if _KS2 < 2**31 else _KS2 - 2**32))

_LT = [2.81022636e-08, 3.43273939e-07, -3.5233877e-06, -4.39150654e-06,
       0.00021858087, -0.00125372503, -0.00417768164, 0.246640727, 1.50140941]
_GT = [-0.000200214257, 0.000100950558, 0.00134934322, -0.00367342844,
       0.00573950773, -0.0076224613, 0.00943887047, 1.00167406, 2.83297682]

_LANES = 128
_ROWS = 1600


def _rotl(x, r):
    return lax.bitwise_or(lax.shift_left(x, np.int32(r)),
                          lax.shift_right_logical(x, np.int32(32 - r)))


def _noise_body(theta_ref, out_ref):
    g = pl.program_id(0)
    base = g * np.int32(_ROWS * _LANES)
    r_iota = lax.broadcasted_iota(jnp.int32, (_ROWS, _LANES), 0)
    c_iota = lax.broadcasted_iota(jnp.int32, (_ROWS, _LANES), 1)
    x1 = base + r_iota * np.int32(_LANES) + c_iota + np.int32(42)
    x0 = jnp.zeros((_ROWS, _LANES), jnp.int32)
    for t in range(5):
        for r in (_ROT_A if t % 2 == 0 else _ROT_B):
            x0 = x0 + x1
            x1 = _rotl(x1, r)
            x1 = lax.bitwise_xor(x0, x1)
        x0 = x0 + _KS[(t + 1) % 3]
        x1 = x1 + (_KS[(t + 2) % 3] + np.int32(t + 1))
    bits = lax.bitwise_xor(x0, x1)
    f = lax.bitcast_convert_type(
        lax.bitwise_or(lax.shift_right_logical(bits, np.int32(9)),
                       np.int32(0x3F800000)), jnp.float32) - np.float32(1.0)
    lo = np.float32(-0.9999999403953552)
    span = np.float32(1.9999999403953552)
    u = jnp.maximum(lo, f * span + lo)
    w = -jnp.log1p(-u * u)
    w1 = w - np.float32(2.5)
    w2 = jnp.sqrt(w) - np.float32(3.0)
    p1 = jnp.full((_ROWS, _LANES), _LT[0], jnp.float32)
    for c in _LT[1:]:
        p1 = p1 * w1 + np.float32(c)
    p2 = jnp.full((_ROWS, _LANES), _GT[0], jnp.float32)
    for c in _GT[1:]:
        p2 = p2 * w2 + np.float32(c)
    z = jnp.where(w < np.float32(5.0), p1, p2) * u * np.float32(1.4142135381698608)
    sig = jnp.exp(theta_ref[...])
    a = sig[:, 0:1]
    b = sig[:, 1:2]
    out_ref[:, 0:64] = z[:, 0:64] * a
    out_ref[:, 64:128] = z[:, 64:128] * b


def kernel(concepts, embeddings, thetas_table):
    B, L = concepts.shape
    D = embeddings.shape[-1]
    n = B * L * D
    n_rows = n // _LANES
    grid = n_rows // _ROWS
    thetas = jnp.take(thetas_table[:, 0], concepts, axis=0)
    thetas2 = thetas.reshape(n_rows, 2)
    out2 = pl.pallas_call(
        _noise_body,
        grid=(grid,),
        in_specs=[pl.BlockSpec((_ROWS, 2), lambda i: (i, 0))],
        out_specs=pl.BlockSpec((_ROWS, _LANES), lambda i: (i, 0)),
        out_shape=jax.ShapeDtypeStruct((n_rows, _LANES), jnp.float32),
    )(thetas2)
    return out2.reshape(B, L, D)


# speedup vs baseline: 1.7109x; 1.0964x over previous
import numpy as np
import jax
import jax.numpy as jnp
from jax import lax
from jax.experimental import pallas as pl
from jax.experimental.pallas import tpu as pltpu

_ROT_A = (13, 15, 26, 6)
_ROT_B = (17, 29, 16, 24)
_KS2 = 466688986 ^ 42
_KS = (np.int32(0), np.int32(42), np.int32(_KS2 if _KS2 < 2**31 else _KS2 - 2**32))

_LT = [2.81022636e-08, 3.43273939e-07, -3.5233877e-06, -4.39150654e-06,
       0.00021858087, -0.00125372503, -0.00417768164, 0.246640727, 1.50140941]
_GT = [-0.000200214257, 0.000100950558, 0.00134934322, -0.00367342844,
       0.00573950773, -0.0076224613, 0.00943887047, 1.00167406, 2.83297682]

_HB = 8
_L = 200
_D = 64
_ROWS = _HB * _L


def _rotl(x, r):
    return lax.bitwise_or(lax.shift_left(x, np.int32(r)),
                          lax.shift_right_logical(x, np.int32(32 - r)))


def _noise_body(theta_ref, out_ref):
    g = pl.program_id(0)
    shp = (_ROWS, 128)
    half = np.int32(_ROWS * _D)
    base_a = g * np.int32(2 * _ROWS * _D)
    r_iota = lax.broadcasted_iota(jnp.int32, shp, 0)
    c_iota = lax.broadcasted_iota(jnp.int32, shp, 1)
    in_b = c_iota >= np.int32(_D)
    i = base_a + r_iota * np.int32(_D) + c_iota + jnp.where(
        in_b, half - np.int32(_D), np.int32(0))
    x1 = i + np.int32(42)
    x0 = jnp.zeros(shp, jnp.int32)
    for t in range(5):
        for r in (_ROT_A if t % 2 == 0 else _ROT_B):
            x0 = x0 + x1
            x1 = _rotl(x1, r)
            x1 = lax.bitwise_xor(x0, x1)
        x0 = x0 + _KS[(t + 1) % 3]
        x1 = x1 + (_KS[(t + 2) % 3] + np.int32(t + 1))
    bits = lax.bitwise_xor(x0, x1)
    f = lax.bitcast_convert_type(
        lax.bitwise_or(lax.shift_right_logical(bits, np.int32(9)),
                       np.int32(0x3F800000)), jnp.float32) - np.float32(1.0)
    lo = np.float32(-0.9999999403953552)
    span = np.float32(1.9999999403953552)
    u = jnp.maximum(lo, f * span + lo)
    w = -jnp.log1p(-u * u)
    w1 = w - np.float32(2.5)
    w2 = jnp.sqrt(w) - np.float32(3.0)
    p1 = jnp.full(shp, _LT[0], jnp.float32)
    for c in _LT[1:]:
        p1 = p1 * w1 + np.float32(c)
    p2 = jnp.full(shp, _GT[0], jnp.float32)
    for c in _GT[1:]:
        p2 = p2 * w2 + np.float32(c)
    z = jnp.where(w < np.float32(5.0), p1, p2) * u * np.float32(1.4142135381698608)
    sig = jnp.exp(theta_ref[0])
    zs = z * jnp.where(in_b, sig[:, 1:2], sig[:, 0:1])
    out_ref[...] = jnp.concatenate(
        [zs[:, 0:_D].reshape(_HB, _L, _D), zs[:, _D:128].reshape(_HB, _L, _D)],
        axis=0)


def kernel(concepts, embeddings, thetas_table):
    B, L = concepts.shape
    D = embeddings.shape[-1]
    grid = B // (2 * _HB)
    thetas = jnp.take(thetas_table[:, 0], concepts, axis=0)
    thetas_t = thetas.reshape(grid, 2, _ROWS).transpose(0, 2, 1)
    out = pl.pallas_call(
        _noise_body,
        grid=(grid,),
        in_specs=[pl.BlockSpec((1, _ROWS, 2), lambda i: (i, 0, 0))],
        out_specs=pl.BlockSpec((2 * _HB, L, D), lambda i: (i, 0, 0)),
        out_shape=jax.ShapeDtypeStruct((B, L, D), jnp.float32),
    )(thetas_t)
    return out
